# baseline (speedup 1.0000x reference)
import functools

import jax
import jax.numpy as jnp
from jax.experimental import pallas as pl
from jax.experimental.pallas import tpu as pltpu
from jax.experimental.pallas import tpu_sc as plsc

_EPS = 1e-12
_GATHER_W = 128
_BBL = 128
_SC = 40


def _sc_gather(table_pad, idx_flat):
    n = idx_flat.shape[0]
    h2 = table_pad.shape[1]
    mesh = plsc.VectorSubcoreMesh(core_axis_name="c", subcore_axis_name="s")
    idx2 = idx_flat.reshape(1, n)

    sub = 2
    w = sub * _GATHER_W

    @functools.partial(
        pl.kernel,
        out_type=jax.ShapeDtypeStruct((n, h2), jnp.bfloat16),
        mesh=mesh,
        scratch_types=[pltpu.SemaphoreType.DMA],
        compiler_params=pltpu.CompilerParams(use_tc_tiling_on_sc=False),
    )
    def gather_kernel(table_hbm, i_hbm, o_hbm, sem):
        def body(i_vmem, o_vmem):
            cps = []
            for t in range(sub):
                cps.append(pltpu.async_copy(
                    table_hbm.at[i_vmem.at[0, pl.ds(t * _GATHER_W, _GATHER_W)]],
                    o_vmem.at[pl.ds(t * _GATHER_W, _GATHER_W), :],
                    sem,
                ))
            for cp in cps:
                cp.wait()

        pltpu.emit_pipeline(
            body,
            grid=(n // w,),
            in_specs=[pl.BlockSpec((1, w), lambda i: (0, i))],
            out_specs=[pl.BlockSpec((w, h2), lambda i: (i, 0))],
            core_axis_name=("c", "s"),
            dimension_semantics=(pltpu.PARALLEL,),
        )(i_hbm, o_hbm)

    return gather_kernel(table_pad, idx2)


def _tr_body(in_ref, out_ref):
    h = in_ref.shape[0]
    out_ref[:, :h] = in_ref[...].T.astype(jnp.bfloat16)


def _tc_transpose_pad(table_t):
    h, v = table_t.shape
    c = 16384
    return pl.pallas_call(
        _tr_body,
        grid=((v + c - 1) // c,),
        in_specs=[pl.BlockSpec((h, c), lambda i: (0, i))],
        out_specs=pl.BlockSpec((c, 2 * h), lambda i: (i, 0)),
        out_shape=jax.ShapeDtypeStruct((v, 2 * h), jnp.bfloat16),
    )(table_t)


def _ln_body(wemb_ref, ttt_ref, pos_ref, ttab_ref, gamma_ref, beta_ref, out_ref):
    s, h, bbl = out_ref.shape
    x2 = wemb_ref[:, :h]
    x = x2.astype(jnp.float32).reshape(bbl, s, h)
    xt = jnp.transpose(x, (1, 2, 0))
    tt = ttt_ref[...].astype(jnp.float32)
    pos = pos_ref[...][:, :, None]
    ttab = ttab_ref[...]
    t0 = ttab[0:1, :][:, :, None]
    dt = ttab[1:2, :][:, :, None] - t0
    emb = xt + pos + t0 + tt[:, None, :] * dt
    mean = jnp.mean(emb, axis=1, keepdims=True)
    cen = emb - mean
    var = jnp.mean(cen * cen, axis=1, keepdims=True)
    normed = cen * jax.lax.rsqrt(var + _EPS)
    gamma = gamma_ref[0:1, :][:, :, None]
    beta = beta_ref[0:1, :][:, :, None]
    out_ref[...] = normed * gamma + beta


def _tc_layernorm(wemb2d, b, s, token_type_ids, pos_s, type_table, gamma, beta):
    h = wemb2d.shape[1] // 2
    tt_t = jnp.swapaxes(token_type_ids, 0, 1)
    ttab = jnp.pad(type_table, ((0, 6), (0, 0)))
    gamma8 = jnp.pad(gamma.reshape(1, h), ((0, 7), (0, 0)))
    beta8 = jnp.pad(beta.reshape(1, h), ((0, 7), (0, 0)))
    grid = (b // _BBL,)
    out_t = pl.pallas_call(
        _ln_body,
        grid=grid,
        in_specs=[
            pl.BlockSpec((_BBL * s, 2 * h), lambda i: (i, 0)),
            pl.BlockSpec((s, _BBL), lambda i: (0, i)),
            pl.BlockSpec((s, h), lambda i: (0, 0)),
            pl.BlockSpec((8, h), lambda i: (0, 0)),
            pl.BlockSpec((8, h), lambda i: (0, 0)),
            pl.BlockSpec((8, h), lambda i: (0, 0)),
        ],
        out_specs=pl.BlockSpec((s, h, _BBL), lambda i: (0, 0, i)),
        out_shape=jax.ShapeDtypeStruct((s, h, b), jnp.float32),
        compiler_params=pltpu.CompilerParams(vmem_limit_bytes=50 * 2**20),
    )(wemb2d, tt_t, pos_s, ttab, gamma8, beta8)
    return jnp.transpose(out_t, (2, 0, 1))


def kernel(input_ids, token_type_ids, word_table, pos_table, type_table, gamma, beta):
    b, s = input_ids.shape
    h = word_table.shape[1]
    table_pad = _tc_transpose_pad(jnp.swapaxes(word_table, 0, 1))
    pos_s = pos_table[:s]
    wemb2d = _sc_gather(table_pad, input_ids.reshape(-1))
    return _tc_layernorm(wemb2d, b, s, token_type_ids, pos_s, type_table,
                         gamma, beta)


# speedup vs baseline: 2.9165x; 2.9165x over previous
import functools

import jax
import jax.numpy as jnp
from jax.experimental import pallas as pl
from jax.experimental.pallas import tpu as pltpu
from jax.experimental.pallas import tpu_sc as plsc

_EPS = 1e-12
_GATHER_W = 128
_BBL = 256
_SC = 40


def _sc_gather(table_pad, idx_flat):
    n = idx_flat.shape[0]
    h2 = table_pad.shape[1]
    mesh = plsc.VectorSubcoreMesh(core_axis_name="c", subcore_axis_name="s")
    idx2 = idx_flat.reshape(1, n)

    sub = 2
    w = sub * _GATHER_W

    @functools.partial(
        pl.kernel,
        out_type=jax.ShapeDtypeStruct((n, h2), jnp.float32),
        mesh=mesh,
        scratch_types=[pltpu.SemaphoreType.DMA],
        compiler_params=pltpu.CompilerParams(use_tc_tiling_on_sc=False),
    )
    def gather_kernel(table_hbm, i_hbm, o_hbm, sem):
        def body(i_vmem, o_vmem):
            cps = []
            for t in range(sub):
                cps.append(pltpu.async_copy(
                    table_hbm.at[i_vmem.at[0, pl.ds(t * _GATHER_W, _GATHER_W)]],
                    o_vmem.at[pl.ds(t * _GATHER_W, _GATHER_W), :],
                    sem,
                ))
            for cp in cps:
                cp.wait()

        pltpu.emit_pipeline(
            body,
            grid=(n // w,),
            in_specs=[pl.BlockSpec((1, w), lambda i: (0, i))],
            out_specs=[pl.BlockSpec((w, h2), lambda i: (i, 0))],
            core_axis_name=("c", "s"),
            dimension_semantics=(pltpu.PARALLEL,),
        )(i_hbm, o_hbm)

    return gather_kernel(table_pad, idx2)


def _tr_body(in_ref, out_ref):
    h = in_ref.shape[0]
    out_ref[:, :h] = in_ref[...].T


def _tc_transpose_pad(table_t):
    h, v = table_t.shape
    c = 24576
    return pl.pallas_call(
        _tr_body,
        grid=((v + c - 1) // c,),
        in_specs=[pl.BlockSpec((h, c), lambda i: (0, i))],
        out_specs=pl.BlockSpec((c, 2 * h), lambda i: (i, 0)),
        out_shape=jax.ShapeDtypeStruct((v, 2 * h), jnp.float32),
    )(table_t)


def _ln_body(wemb_ref, ttt_ref, pos_ref, ttab_ref, gamma_ref, beta_ref, out_ref):
    h = out_ref.shape[1]
    x = wemb_ref[:, :, :h]
    xt = jnp.transpose(x, (1, 2, 0))
    tt = ttt_ref[...].astype(jnp.float32)
    pos = pos_ref[...][:, :, None]
    ttab = ttab_ref[...]
    dt = (ttab[1:2, :] - ttab[0:1, :])[:, :, None]
    emb = xt + pos + tt[:, None, :] * dt
    mean = jnp.mean(emb, axis=1, keepdims=True)
    cen = emb - mean
    var = jnp.mean(cen * cen, axis=1, keepdims=True)
    normed = cen * jax.lax.rsqrt(var + _EPS)
    gamma = gamma_ref[0:1, :][:, :, None]
    beta = beta_ref[0:1, :][:, :, None]
    out_ref[...] = normed * gamma + beta


def _tc_layernorm(wemb, token_type_ids, pos_s, type_table, gamma, beta):
    b, s, h2 = wemb.shape
    h = h2 // 2
    tt_t = jnp.swapaxes(token_type_ids, 0, 1)
    ttab = jnp.pad(type_table, ((0, 6), (0, 0)))
    gamma8 = jnp.pad(gamma.reshape(1, h), ((0, 7), (0, 0)))
    beta8 = jnp.pad(beta.reshape(1, h), ((0, 7), (0, 0)))
    grid = (b // _BBL, s // _SC)
    out_t = pl.pallas_call(
        _ln_body,
        grid=grid,
        in_specs=[
            pl.BlockSpec((_BBL, _SC, h2), lambda i, j: (i, j, 0)),
            pl.BlockSpec((_SC, _BBL), lambda i, j: (j, i)),
            pl.BlockSpec((_SC, h), lambda i, j: (j, 0)),
            pl.BlockSpec((8, h), lambda i, j: (0, 0)),
            pl.BlockSpec((8, h), lambda i, j: (0, 0)),
            pl.BlockSpec((8, h), lambda i, j: (0, 0)),
        ],
        out_specs=pl.BlockSpec((_SC, h, _BBL), lambda i, j: (j, 0, i)),
        out_shape=jax.ShapeDtypeStruct((s, h, b), jnp.float32),
        compiler_params=pltpu.CompilerParams(vmem_limit_bytes=50 * 2**20),
    )(wemb, tt_t, pos_s, ttab, gamma8, beta8)
    return jnp.transpose(out_t, (2, 0, 1))


def kernel(input_ids, token_type_ids, word_table, pos_table, type_table, gamma, beta):
    b, s = input_ids.shape
    h = word_table.shape[1]
    table_pad = _tc_transpose_pad(jnp.swapaxes(word_table, 0, 1))
    pos_s = pos_table[:s] + type_table[0]
    wemb = _sc_gather(table_pad, input_ids.reshape(-1)).reshape(b, s, 2 * h)
    return _tc_layernorm(wemb, token_type_ids, pos_s, type_table, gamma, beta)


# speedup vs baseline: 3.0960x; 1.0615x over previous
import functools

import jax
import jax.numpy as jnp
from jax.experimental import pallas as pl
from jax.experimental.pallas import tpu as pltpu
from jax.experimental.pallas import tpu_sc as plsc

_EPS = 1e-12
_GATHER_W = 128
_BBL = 256
_SC = 40


def _sc_gather(table_pad, idx_flat):
    n = idx_flat.shape[0]
    h2 = table_pad.shape[1]
    mesh = plsc.VectorSubcoreMesh(core_axis_name="c", subcore_axis_name="s")
    idx2 = idx_flat.reshape(1, n)

    sub = 2
    w = sub * _GATHER_W

    @functools.partial(
        pl.kernel,
        out_type=jax.ShapeDtypeStruct((n, h2), jnp.float32),
        mesh=mesh,
        scratch_types=[pltpu.SemaphoreType.DMA],
        compiler_params=pltpu.CompilerParams(use_tc_tiling_on_sc=False),
    )
    def gather_kernel(table_hbm, i_hbm, o_hbm, sem):
        def body(i_vmem, o_vmem):
            cps = []
            for t in range(sub):
                cps.append(pltpu.async_copy(
                    table_hbm.at[i_vmem.at[0, pl.ds(t * _GATHER_W, _GATHER_W)]],
                    o_vmem.at[pl.ds(t * _GATHER_W, _GATHER_W), :],
                    sem,
                ))
            for cp in cps:
                cp.wait()

        pltpu.emit_pipeline(
            body,
            grid=(n // w,),
            in_specs=[pl.BlockSpec((1, w), lambda i: (0, i))],
            out_specs=[pl.BlockSpec((w, h2), lambda i: (i, 0))],
            core_axis_name=("c", "s"),
            dimension_semantics=(pltpu.PARALLEL,),
        )(i_hbm, o_hbm)

    return gather_kernel(table_pad, idx2)


def _tr_body(in_ref, out_ref):
    h = in_ref.shape[0]
    out_ref[:, :h] = in_ref[...].T


def _tc_transpose_pad(table_t):
    h, v = table_t.shape
    c = 24576
    return pl.pallas_call(
        _tr_body,
        grid=((v + c - 1) // c,),
        in_specs=[pl.BlockSpec((h, c), lambda i: (0, i))],
        out_specs=pl.BlockSpec((c, 2 * h), lambda i: (i, 0)),
        out_shape=jax.ShapeDtypeStruct((v, 2 * h), jnp.float32),
    )(table_t)


def _ln_body_prev(wemb_ref, ttt_ref, pos_ref, ttab_ref, gamma_ref, beta_ref,
                  prev_ref, out_ref):
    del prev_ref
    _ln_body(wemb_ref, ttt_ref, pos_ref, ttab_ref, gamma_ref, beta_ref, out_ref)


def _ln_body(wemb_ref, ttt_ref, pos_ref, ttab_ref, gamma_ref, beta_ref, out_ref):
    h = out_ref.shape[1]
    x = wemb_ref[:, :, :h]
    xt = jnp.transpose(x, (1, 2, 0))
    tt = ttt_ref[...].astype(jnp.float32)
    pos = pos_ref[...][:, :, None]
    ttab = ttab_ref[...]
    dt = (ttab[1:2, :] - ttab[0:1, :])[:, :, None]
    emb = xt + pos + tt[:, None, :] * dt
    mean = jnp.mean(emb, axis=1, keepdims=True)
    cen = emb - mean
    var = jnp.mean(cen * cen, axis=1, keepdims=True)
    normed = cen * jax.lax.rsqrt(var + _EPS)
    gamma = gamma_ref[0:1, :][:, :, None]
    beta = beta_ref[0:1, :][:, :, None]
    out_ref[...] = normed * gamma + beta


def _tc_layernorm_chunk(wemb, tt_t_chunk, pos_s, ttab, gamma8, beta8,
                        b_total, kb, prev):
    bc, s, h2 = wemb.shape
    h = h2 // 2
    grid = (bc // _BBL, s // _SC)
    in_specs = [
        pl.BlockSpec((_BBL, _SC, h2), lambda i, j: (i, j, 0)),
        pl.BlockSpec((_SC, _BBL), lambda i, j: (j, i)),
        pl.BlockSpec((_SC, h), lambda i, j: (j, 0)),
        pl.BlockSpec((8, h), lambda i, j: (0, 0)),
        pl.BlockSpec((8, h), lambda i, j: (0, 0)),
        pl.BlockSpec((8, h), lambda i, j: (0, 0)),
    ]
    args = [wemb, tt_t_chunk, pos_s, ttab, gamma8, beta8]
    body = _ln_body
    aliases = {}
    if prev is not None:
        in_specs.append(pl.BlockSpec(memory_space=pl.ANY))
        args.append(prev)
        body = _ln_body_prev
        aliases = {6: 0}
    return pl.pallas_call(
        body,
        grid=grid,
        in_specs=in_specs,
        out_specs=pl.BlockSpec((_SC, h, _BBL), lambda i, j: (j, 0, kb + i)),
        out_shape=jax.ShapeDtypeStruct((s, h, b_total), jnp.float32),
        input_output_aliases=aliases,
        compiler_params=pltpu.CompilerParams(vmem_limit_bytes=50 * 2**20),
    )(*args)


def kernel(input_ids, token_type_ids, word_table, pos_table, type_table, gamma, beta):
    b, s = input_ids.shape
    h = word_table.shape[1]
    table_pad = _tc_transpose_pad(jnp.swapaxes(word_table, 0, 1))
    pos_s = pos_table[:s] + type_table[0]
    tt_t = jnp.swapaxes(token_type_ids, 0, 1)
    ttab = jnp.pad(type_table, ((0, 6), (0, 0)))
    gamma8 = jnp.pad(gamma.reshape(1, h), ((0, 7), (0, 0)))
    beta8 = jnp.pad(beta.reshape(1, h), ((0, 7), (0, 0)))
    nchunks = 2
    bc = b // nchunks
    out_t = None
    for k in range(nchunks):
        ids_k = jax.lax.slice_in_dim(input_ids, k * bc, (k + 1) * bc, axis=0)
        wemb_k = _sc_gather(table_pad, ids_k.reshape(-1)).reshape(bc, s, 2 * h)
        tt_k = jax.lax.slice_in_dim(tt_t, k * bc, (k + 1) * bc, axis=1)
        out_t = _tc_layernorm_chunk(wemb_k, tt_k, pos_s, ttab, gamma8, beta8,
                                    b, k * (bc // _BBL), out_t)
    return jnp.transpose(out_t, (2, 0, 1))


# speedup vs baseline: 3.1275x; 1.0102x over previous
import functools

import jax
import jax.numpy as jnp
from jax.experimental import pallas as pl
from jax.experimental.pallas import tpu as pltpu
from jax.experimental.pallas import tpu_sc as plsc

_EPS = 1e-12
_GATHER_W = 128
_BBL = 256
_SC = 40


def _sc_gather(table_pad, idx_flat):
    n = idx_flat.shape[0]
    h2 = table_pad.shape[1]
    mesh = plsc.VectorSubcoreMesh(core_axis_name="c", subcore_axis_name="s")
    idx2 = idx_flat.reshape(1, n)

    sub = 2
    w = sub * _GATHER_W

    @functools.partial(
        pl.kernel,
        out_type=jax.ShapeDtypeStruct((n, h2), jnp.float32),
        mesh=mesh,
        scratch_types=[pltpu.SemaphoreType.DMA],
        compiler_params=pltpu.CompilerParams(use_tc_tiling_on_sc=False),
    )
    def gather_kernel(table_hbm, i_hbm, o_hbm, sem):
        def body(i_vmem, o_vmem):
            cps = []
            for t in range(sub):
                cps.append(pltpu.async_copy(
                    table_hbm.at[i_vmem.at[0, pl.ds(t * _GATHER_W, _GATHER_W)]],
                    o_vmem.at[pl.ds(t * _GATHER_W, _GATHER_W), :],
                    sem,
                ))
            for cp in cps:
                cp.wait()

        pltpu.emit_pipeline(
            body,
            grid=(n // w,),
            in_specs=[pl.BlockSpec((1, w), lambda i: (0, i))],
            out_specs=[pl.BlockSpec((w, h2), lambda i: (i, 0))],
            core_axis_name=("c", "s"),
            dimension_semantics=(pltpu.PARALLEL,),
        )(i_hbm, o_hbm)

    return gather_kernel(table_pad, idx2)


def _tr_body(in_ref, out_ref):
    h = in_ref.shape[0]
    out_ref[:, :h] = in_ref[...].T


def _tc_transpose_pad(table_t):
    h, v = table_t.shape
    c = 24576
    return pl.pallas_call(
        _tr_body,
        grid=((v + c - 1) // c,),
        in_specs=[pl.BlockSpec((h, c), lambda i: (0, i))],
        out_specs=pl.BlockSpec((c, 2 * h), lambda i: (i, 0)),
        out_shape=jax.ShapeDtypeStruct((v, 2 * h), jnp.float32),
    )(table_t)


def _ln_body_prev(wemb_ref, ttt_ref, pos_ref, ttab_ref, gamma_ref, beta_ref,
                  prev_ref, out_ref):
    del prev_ref
    _ln_body(wemb_ref, ttt_ref, pos_ref, ttab_ref, gamma_ref, beta_ref, out_ref)


def _ln_body(wemb_ref, ttt_ref, pos_ref, ttab_ref, gamma_ref, beta_ref, out_ref):
    h = out_ref.shape[1]
    x = wemb_ref[:, :, :h]
    xt = jnp.transpose(x, (1, 2, 0))
    tt = ttt_ref[...].astype(jnp.float32)
    pos = pos_ref[...][:, :, None]
    ttab = ttab_ref[...]
    dt = (ttab[1:2, :] - ttab[0:1, :])[:, :, None]
    emb = xt + pos + tt[:, None, :] * dt
    mean = jnp.mean(emb, axis=1, keepdims=True)
    cen = emb - mean
    var = jnp.mean(cen * cen, axis=1, keepdims=True)
    normed = cen * jax.lax.rsqrt(var + _EPS)
    gamma = gamma_ref[0:1, :][:, :, None]
    beta = beta_ref[0:1, :][:, :, None]
    out_ref[...] = normed * gamma + beta


def _tc_layernorm_chunk(wemb, tt_t_chunk, pos_s, ttab, gamma8, beta8,
                        b_total, kb, prev):
    bc, s, h2 = wemb.shape
    h = h2 // 2
    grid = (bc // _BBL, s // _SC)
    in_specs = [
        pl.BlockSpec((_BBL, _SC, h2), lambda i, j: (i, j, 0)),
        pl.BlockSpec((_SC, _BBL), lambda i, j: (j, i)),
        pl.BlockSpec((_SC, h), lambda i, j: (j, 0)),
        pl.BlockSpec((8, h), lambda i, j: (0, 0)),
        pl.BlockSpec((8, h), lambda i, j: (0, 0)),
        pl.BlockSpec((8, h), lambda i, j: (0, 0)),
    ]
    args = [wemb, tt_t_chunk, pos_s, ttab, gamma8, beta8]
    body = _ln_body
    aliases = {}
    if prev is not None:
        in_specs.append(pl.BlockSpec(memory_space=pl.ANY))
        args.append(prev)
        body = _ln_body_prev
        aliases = {6: 0}
    return pl.pallas_call(
        body,
        grid=grid,
        in_specs=in_specs,
        out_specs=pl.BlockSpec((_SC, h, _BBL), lambda i, j: (j, 0, kb + i)),
        out_shape=jax.ShapeDtypeStruct((s, h, b_total), jnp.float32),
        input_output_aliases=aliases,
        compiler_params=pltpu.CompilerParams(vmem_limit_bytes=50 * 2**20),
    )(*args)


def kernel(input_ids, token_type_ids, word_table, pos_table, type_table, gamma, beta):
    b, s = input_ids.shape
    h = word_table.shape[1]
    table_pad = _tc_transpose_pad(jnp.swapaxes(word_table, 0, 1))
    pos_s = pos_table[:s] + type_table[0]
    tt_t = jnp.swapaxes(token_type_ids, 0, 1)
    ttab = jnp.pad(type_table, ((0, 6), (0, 0)))
    gamma8 = jnp.pad(gamma.reshape(1, h), ((0, 7), (0, 0)))
    beta8 = jnp.pad(beta.reshape(1, h), ((0, 7), (0, 0)))
    nchunks = 4
    bc = b // nchunks
    out_t = None
    for k in range(nchunks):
        ids_k = jax.lax.slice_in_dim(input_ids, k * bc, (k + 1) * bc, axis=0)
        wemb_k = _sc_gather(table_pad, ids_k.reshape(-1)).reshape(bc, s, 2 * h)
        tt_k = jax.lax.slice_in_dim(tt_t, k * bc, (k + 1) * bc, axis=1)
        out_t = _tc_layernorm_chunk(wemb_k, tt_k, pos_s, ttab, gamma8, beta8,
                                    b, k * (bc // _BBL), out_t)
    return jnp.transpose(out_t, (2, 0, 1))


# speedup vs baseline: 3.1402x; 1.0041x over previous
import functools

import jax
import jax.numpy as jnp
from jax.experimental import pallas as pl
from jax.experimental.pallas import tpu as pltpu
from jax.experimental.pallas import tpu_sc as plsc

_EPS = 1e-12
_GATHER_W = 128
_BBL = 256
_SC = 40


def _sc_gather(table_pad, idx_flat):
    n = idx_flat.shape[0]
    h2 = table_pad.shape[1]
    mesh = plsc.VectorSubcoreMesh(core_axis_name="c", subcore_axis_name="s")
    idx2 = idx_flat.reshape(1, n)

    sub = 2
    w = sub * _GATHER_W

    @functools.partial(
        pl.kernel,
        out_type=jax.ShapeDtypeStruct((n, h2), jnp.float32),
        mesh=mesh,
        scratch_types=[pltpu.SemaphoreType.DMA],
        compiler_params=pltpu.CompilerParams(use_tc_tiling_on_sc=False),
    )
    def gather_kernel(table_hbm, i_hbm, o_hbm, sem):
        def body(i_vmem, o_vmem):
            cps = []
            for t in range(sub):
                cps.append(pltpu.async_copy(
                    table_hbm.at[i_vmem.at[0, pl.ds(t * _GATHER_W, _GATHER_W)]],
                    o_vmem.at[pl.ds(t * _GATHER_W, _GATHER_W), :],
                    sem,
                ))
            for cp in cps:
                cp.wait()

        pltpu.emit_pipeline(
            body,
            grid=(n // w,),
            in_specs=[pl.BlockSpec((1, w), lambda i: (0, i))],
            out_specs=[pl.BlockSpec((w, h2), lambda i: (i, 0))],
            core_axis_name=("c", "s"),
            dimension_semantics=(pltpu.PARALLEL,),
        )(i_hbm, o_hbm)

    return gather_kernel(table_pad, idx2)


def _tr_body(in_ref, out_ref):
    h = in_ref.shape[0]
    out_ref[:, :h] = in_ref[...].T


def _tc_transpose_pad(table_t):
    h, v = table_t.shape
    c = 32768
    return pl.pallas_call(
        _tr_body,
        grid=((v + c - 1) // c,),
        in_specs=[pl.BlockSpec((h, c), lambda i: (0, i))],
        out_specs=pl.BlockSpec((c, 2 * h), lambda i: (i, 0)),
        out_shape=jax.ShapeDtypeStruct((v, 2 * h), jnp.float32),
    )(table_t)


def _ln_body_prev(wemb_ref, ttt_ref, pos_ref, ttab_ref, gamma_ref, beta_ref,
                  prev_ref, out_ref):
    del prev_ref
    _ln_body(wemb_ref, ttt_ref, pos_ref, ttab_ref, gamma_ref, beta_ref, out_ref)


def _ln_body(wemb_ref, ttt_ref, pos_ref, ttab_ref, gamma_ref, beta_ref, out_ref):
    h = out_ref.shape[1]
    x = wemb_ref[:, :, :h]
    xt = jnp.transpose(x, (1, 2, 0))
    tt = ttt_ref[...].astype(jnp.float32)
    pos = pos_ref[...][:, :, None]
    ttab = ttab_ref[...]
    dt = (ttab[1:2, :] - ttab[0:1, :])[:, :, None]
    emb = xt + pos + tt[:, None, :] * dt
    mean = jnp.mean(emb, axis=1, keepdims=True)
    cen = emb - mean
    var = jnp.mean(cen * cen, axis=1, keepdims=True)
    normed = cen * jax.lax.rsqrt(var + _EPS)
    gamma = gamma_ref[0:1, :][:, :, None]
    beta = beta_ref[0:1, :][:, :, None]
    out_ref[...] = normed * gamma + beta


def _tc_layernorm_chunk(wemb, tt_t_chunk, pos_s, ttab, gamma8, beta8,
                        b_total, kb, prev):
    bc, s, h2 = wemb.shape
    h = h2 // 2
    grid = (bc // _BBL, s // _SC)
    in_specs = [
        pl.BlockSpec((_BBL, _SC, h2), lambda i, j: (i, j, 0)),
        pl.BlockSpec((_SC, _BBL), lambda i, j: (j, i)),
        pl.BlockSpec((_SC, h), lambda i, j: (j, 0)),
        pl.BlockSpec((8, h), lambda i, j: (0, 0)),
        pl.BlockSpec((8, h), lambda i, j: (0, 0)),
        pl.BlockSpec((8, h), lambda i, j: (0, 0)),
    ]
    args = [wemb, tt_t_chunk, pos_s, ttab, gamma8, beta8]
    body = _ln_body
    aliases = {}
    if prev is not None:
        in_specs.append(pl.BlockSpec(memory_space=pl.ANY))
        args.append(prev)
        body = _ln_body_prev
        aliases = {6: 0}
    return pl.pallas_call(
        body,
        grid=grid,
        in_specs=in_specs,
        out_specs=pl.BlockSpec((_SC, h, _BBL), lambda i, j: (j, 0, kb + i)),
        out_shape=jax.ShapeDtypeStruct((s, h, b_total), jnp.float32),
        input_output_aliases=aliases,
        compiler_params=pltpu.CompilerParams(vmem_limit_bytes=50 * 2**20),
    )(*args)


def kernel(input_ids, token_type_ids, word_table, pos_table, type_table, gamma, beta):
    b, s = input_ids.shape
    h = word_table.shape[1]
    table_pad = _tc_transpose_pad(jnp.swapaxes(word_table, 0, 1))
    pos_s = pos_table[:s] + type_table[0]
    tt_t = jnp.swapaxes(token_type_ids, 0, 1)
    ttab = jnp.pad(type_table, ((0, 6), (0, 0)))
    gamma8 = jnp.pad(gamma.reshape(1, h), ((0, 7), (0, 0)))
    beta8 = jnp.pad(beta.reshape(1, h), ((0, 7), (0, 0)))
    nchunks = 4
    bc = b // nchunks
    out_t = None
    for k in range(nchunks):
        ids_k = jax.lax.slice_in_dim(input_ids, k * bc, (k + 1) * bc, axis=0)
        wemb_k = _sc_gather(table_pad, ids_k.reshape(-1)).reshape(bc, s, 2 * h)
        tt_k = jax.lax.slice_in_dim(tt_t, k * bc, (k + 1) * bc, axis=1)
        out_t = _tc_layernorm_chunk(wemb_k, tt_k, pos_s, ttab, gamma8, beta8,
                                    b, k * (bc // _BBL), out_t)
    return jnp.transpose(out_t, (2, 0, 1))
